# Initial kernel scaffold; baseline (speedup 1.0000x reference)
import jax
import jax.numpy as jnp
from jax.experimental import pallas as pl


def kernel(z_fast, z_slow, cb_syn, cb_sem, syn_W1, syn_b1, syn_g1, syn_be1, syn_W2, syn_b2, syn_Wp, syn_bp, sem_W1, sem_b1, sem_g1, sem_be1, sem_W2, sem_b2, sem_Wp, sem_bp, update_graph):
    raise NotImplementedError("write your pallas kernel here")


# speedup vs baseline: 1.6125x; 1.6125x over previous
import functools

import jax
import jax.numpy as jnp
from jax.experimental import pallas as pl

N = 32768
DIM = 64
N_SYN = 512
N_SEM = 1024
CTX = 3.0
COMMIT = 0.25
LN_EPS = 1e-5

BLK = 256


def _ln_rows(x):
    m = jnp.mean(x, axis=-1, keepdims=True)
    v = jnp.mean((x - m) ** 2, axis=-1, keepdims=True)
    return (x - m) / jnp.sqrt(v + LN_EPS)


def _branch(z, cb, W1, b1, g1, be1, W2, b2, Wp, bp, k):
    h = jnp.dot(z, W1, preferred_element_type=jnp.float32) + b1
    h = _ln_rows(h) * g1 + be1
    h = jnp.maximum(h, 0.0)
    h = jnp.maximum(jnp.dot(h, W2, preferred_element_type=jnp.float32) + b2, 0.0)
    ctx = _ln_rows(jnp.dot(h, Wp, preferred_element_type=jnp.float32) + bp)

    zsq = jnp.sum(z * z, axis=1, keepdims=True)
    csq = jnp.sum(cb * cb, axis=1)[None, :]
    d = zsq + csq - 2.0 * jnp.dot(z, cb.T, preferred_element_type=jnp.float32)
    logits = _ln_rows(-jnp.clip(d, 0.0, 10000.0))

    total = logits + CTX * ctx
    idx = jnp.argmax(total, axis=1)
    idx_pure = jnp.argmax(logits, axis=1)

    onehot = (jax.lax.broadcasted_iota(jnp.int32, (z.shape[0], k), 1)
              == idx[:, None]).astype(jnp.float32)
    zq = jnp.dot(onehot, cb, preferred_element_type=jnp.float32)
    loss = (1.0 + COMMIT) * jnp.mean((zq - z) ** 2, axis=1)
    flags = (idx != idx_pure).astype(jnp.float32)
    divpart = jnp.sum(flags.reshape(-1, 128), axis=0)[None, :]
    return zq, loss, idx.astype(jnp.int32), divpart


def _vq_kernel(zf_ref, zs_ref, cbsyn_ref, cbsem_ref,
               syn_W1, syn_b1, syn_g1, syn_be1, syn_W2, syn_b2, syn_Wp, syn_bp,
               sem_W1, sem_b1, sem_g1, sem_be1, sem_W2, sem_b2, sem_Wp, sem_bp,
               zq_syn_ref, zq_sem_ref, loss_syn_ref, loss_sem_ref,
               idx_syn_ref, idx_sem_ref, div_syn_ref, div_sem_ref):
    zq_s, loss_s, idx_s, cnt_s = _branch(
        zf_ref[...], cbsyn_ref[...],
        syn_W1[...], syn_b1[...], syn_g1[...], syn_be1[...],
        syn_W2[...], syn_b2[...], syn_Wp[...], syn_bp[...], N_SYN)
    zq_m, loss_m, idx_m, cnt_m = _branch(
        zs_ref[...], cbsem_ref[...],
        sem_W1[...], sem_b1[...], sem_g1[...], sem_be1[...],
        sem_W2[...], sem_b2[...], sem_Wp[...], sem_bp[...], N_SEM)

    zq_syn_ref[...] = zq_s
    zq_sem_ref[...] = zq_m
    loss_syn_ref[...] = loss_s
    loss_sem_ref[...] = loss_m
    idx_syn_ref[...] = idx_s
    idx_sem_ref[...] = idx_m

    @pl.when(pl.program_id(0) == 0)
    def _():
        div_syn_ref[...] = jnp.zeros_like(div_syn_ref)
        div_sem_ref[...] = jnp.zeros_like(div_sem_ref)

    div_syn_ref[...] += cnt_s * (1.0 / N)
    div_sem_ref[...] += cnt_m * (1.0 / N)


def kernel(z_fast, z_slow, cb_syn, cb_sem,
           syn_W1, syn_b1, syn_g1, syn_be1, syn_W2, syn_b2, syn_Wp, syn_bp,
           sem_W1, sem_b1, sem_g1, sem_be1, sem_W2, sem_b2, sem_Wp, sem_bp,
           update_graph):
    grid = (N // BLK,)
    row_spec = pl.BlockSpec((BLK, DIM), lambda i: (i, 0))
    full = lambda shape: pl.BlockSpec(shape, lambda i: (0,) * len(shape))

    in_specs = [
        row_spec, row_spec,
        full((N_SYN, DIM)), full((N_SEM, DIM)),
        full((DIM, DIM)), full((DIM,)), full((DIM,)), full((DIM,)),
        full((DIM, DIM // 2)), full((DIM // 2,)),
        full((DIM // 2, N_SYN)), full((N_SYN,)),
        full((DIM, DIM)), full((DIM,)), full((DIM,)), full((DIM,)),
        full((DIM, DIM // 2)), full((DIM // 2,)),
        full((DIM // 2, N_SEM)), full((N_SEM,)),
    ]
    out_specs = [
        row_spec, row_spec,
        pl.BlockSpec((BLK,), lambda i: (i,)),
        pl.BlockSpec((BLK,), lambda i: (i,)),
        pl.BlockSpec((BLK,), lambda i: (i,)),
        pl.BlockSpec((BLK,), lambda i: (i,)),
        pl.BlockSpec((1, 128), lambda i: (0, 0)),
        pl.BlockSpec((1, 128), lambda i: (0, 0)),
    ]
    out_shapes = [
        jax.ShapeDtypeStruct((N, DIM), jnp.float32),
        jax.ShapeDtypeStruct((N, DIM), jnp.float32),
        jax.ShapeDtypeStruct((N,), jnp.float32),
        jax.ShapeDtypeStruct((N,), jnp.float32),
        jax.ShapeDtypeStruct((N,), jnp.int32),
        jax.ShapeDtypeStruct((N,), jnp.int32),
        jax.ShapeDtypeStruct((1, 128), jnp.float32),
        jax.ShapeDtypeStruct((1, 128), jnp.float32),
    ]

    (zq_syn, zq_sem, loss_syn, loss_sem, idx_syn, idx_sem,
     div_syn, div_sem) = pl.pallas_call(
        _vq_kernel,
        grid=grid,
        in_specs=in_specs,
        out_specs=out_specs,
        out_shape=out_shapes,
    )(z_fast, z_slow, cb_syn, cb_sem,
      syn_W1, syn_b1, syn_g1, syn_be1, syn_W2, syn_b2, syn_Wp, syn_bp,
      sem_W1, sem_b1, sem_g1, sem_be1, sem_W2, sem_b2, sem_Wp, sem_bp)

    return (zq_syn, zq_sem, loss_syn, loss_sem, idx_syn, idx_sem,
            jnp.sum(div_syn), jnp.sum(div_sem))


# speedup vs baseline: 1.7618x; 1.0926x over previous
import functools

import jax
import jax.numpy as jnp
from jax.experimental import pallas as pl

N = 32768
DIM = 64
N_SYN = 512
N_SEM = 1024
CTX = 3.0
COMMIT = 0.25
LN_EPS = 1e-5

BLK = 512


def _ln_rows(x):
    m = jnp.mean(x, axis=-1, keepdims=True)
    v = jnp.mean((x - m) ** 2, axis=-1, keepdims=True)
    return (x - m) / jnp.sqrt(v + LN_EPS)


def _branch(z, cb, W1, b1, g1, be1, W2, b2, Wp, bp, k):
    h = jnp.dot(z, W1, preferred_element_type=jnp.float32) + b1
    h = _ln_rows(h) * g1 + be1
    h = jnp.maximum(h, 0.0)
    h = jnp.maximum(jnp.dot(h, W2, preferred_element_type=jnp.float32) + b2, 0.0)
    ctx = _ln_rows(jnp.dot(h, Wp, preferred_element_type=jnp.float32) + bp)

    zsq = jnp.sum(z * z, axis=1, keepdims=True)
    csq = jnp.sum(cb * cb, axis=1)[None, :]
    d = zsq + csq - 2.0 * jnp.dot(z, cb.T, preferred_element_type=jnp.float32)
    logits = _ln_rows(-jnp.clip(d, 0.0, 10000.0))

    total = logits + CTX * ctx
    idx = jnp.argmax(total, axis=1)
    idx_pure = jnp.argmax(logits, axis=1)

    onehot = (jax.lax.broadcasted_iota(jnp.int32, (z.shape[0], k), 1)
              == idx[:, None]).astype(jnp.float32)
    zq = jnp.dot(onehot, cb, preferred_element_type=jnp.float32)
    loss = (1.0 + COMMIT) * jnp.mean((zq - z) ** 2, axis=1)
    flags = (idx != idx_pure).astype(jnp.float32)
    divpart = jnp.sum(flags.reshape(-1, 128), axis=0)[None, :]
    return zq, loss, idx.astype(jnp.int32), divpart


def _vq_kernel(zf_ref, zs_ref, cbsyn_ref, cbsem_ref,
               syn_W1, syn_b1, syn_g1, syn_be1, syn_W2, syn_b2, syn_Wp, syn_bp,
               sem_W1, sem_b1, sem_g1, sem_be1, sem_W2, sem_b2, sem_Wp, sem_bp,
               zq_syn_ref, zq_sem_ref, loss_syn_ref, loss_sem_ref,
               idx_syn_ref, idx_sem_ref, div_syn_ref, div_sem_ref):
    zq_s, loss_s, idx_s, cnt_s = _branch(
        zf_ref[...], cbsyn_ref[...],
        syn_W1[...], syn_b1[...], syn_g1[...], syn_be1[...],
        syn_W2[...], syn_b2[...], syn_Wp[...], syn_bp[...], N_SYN)
    zq_m, loss_m, idx_m, cnt_m = _branch(
        zs_ref[...], cbsem_ref[...],
        sem_W1[...], sem_b1[...], sem_g1[...], sem_be1[...],
        sem_W2[...], sem_b2[...], sem_Wp[...], sem_bp[...], N_SEM)

    zq_syn_ref[...] = zq_s
    zq_sem_ref[...] = zq_m
    loss_syn_ref[...] = loss_s
    loss_sem_ref[...] = loss_m
    idx_syn_ref[...] = idx_s
    idx_sem_ref[...] = idx_m

    @pl.when(pl.program_id(0) == 0)
    def _():
        div_syn_ref[...] = jnp.zeros_like(div_syn_ref)
        div_sem_ref[...] = jnp.zeros_like(div_sem_ref)

    div_syn_ref[...] += cnt_s * (1.0 / N)
    div_sem_ref[...] += cnt_m * (1.0 / N)


def kernel(z_fast, z_slow, cb_syn, cb_sem,
           syn_W1, syn_b1, syn_g1, syn_be1, syn_W2, syn_b2, syn_Wp, syn_bp,
           sem_W1, sem_b1, sem_g1, sem_be1, sem_W2, sem_b2, sem_Wp, sem_bp,
           update_graph):
    grid = (N // BLK,)
    row_spec = pl.BlockSpec((BLK, DIM), lambda i: (i, 0))
    full = lambda shape: pl.BlockSpec(shape, lambda i: (0,) * len(shape))

    in_specs = [
        row_spec, row_spec,
        full((N_SYN, DIM)), full((N_SEM, DIM)),
        full((DIM, DIM)), full((DIM,)), full((DIM,)), full((DIM,)),
        full((DIM, DIM // 2)), full((DIM // 2,)),
        full((DIM // 2, N_SYN)), full((N_SYN,)),
        full((DIM, DIM)), full((DIM,)), full((DIM,)), full((DIM,)),
        full((DIM, DIM // 2)), full((DIM // 2,)),
        full((DIM // 2, N_SEM)), full((N_SEM,)),
    ]
    out_specs = [
        row_spec, row_spec,
        pl.BlockSpec((BLK,), lambda i: (i,)),
        pl.BlockSpec((BLK,), lambda i: (i,)),
        pl.BlockSpec((BLK,), lambda i: (i,)),
        pl.BlockSpec((BLK,), lambda i: (i,)),
        pl.BlockSpec((1, 128), lambda i: (0, 0)),
        pl.BlockSpec((1, 128), lambda i: (0, 0)),
    ]
    out_shapes = [
        jax.ShapeDtypeStruct((N, DIM), jnp.float32),
        jax.ShapeDtypeStruct((N, DIM), jnp.float32),
        jax.ShapeDtypeStruct((N,), jnp.float32),
        jax.ShapeDtypeStruct((N,), jnp.float32),
        jax.ShapeDtypeStruct((N,), jnp.int32),
        jax.ShapeDtypeStruct((N,), jnp.int32),
        jax.ShapeDtypeStruct((1, 128), jnp.float32),
        jax.ShapeDtypeStruct((1, 128), jnp.float32),
    ]

    (zq_syn, zq_sem, loss_syn, loss_sem, idx_syn, idx_sem,
     div_syn, div_sem) = pl.pallas_call(
        _vq_kernel,
        grid=grid,
        in_specs=in_specs,
        out_specs=out_specs,
        out_shape=out_shapes,
    )(z_fast, z_slow, cb_syn, cb_sem,
      syn_W1, syn_b1, syn_g1, syn_be1, syn_W2, syn_b2, syn_Wp, syn_bp,
      sem_W1, sem_b1, sem_g1, sem_be1, sem_W2, sem_b2, sem_Wp, sem_bp)

    return (zq_syn, zq_sem, loss_syn, loss_sem, idx_syn, idx_sem,
            jnp.sum(div_syn), jnp.sum(div_sem))


# speedup vs baseline: 1.7737x; 1.0068x over previous
import functools

import jax
import jax.numpy as jnp
from jax.experimental import pallas as pl

N = 32768
DIM = 64
N_SYN = 512
N_SEM = 1024
CTX = 3.0
COMMIT = 0.25
LN_EPS = 1e-5

BLK = 1024


def _ln_rows(x):
    m = jnp.mean(x, axis=-1, keepdims=True)
    v = jnp.mean((x - m) ** 2, axis=-1, keepdims=True)
    return (x - m) / jnp.sqrt(v + LN_EPS)


def _branch(z, cb, W1, b1, g1, be1, W2, b2, Wp, bp, k):
    h = jnp.dot(z, W1, preferred_element_type=jnp.float32) + b1
    h = _ln_rows(h) * g1 + be1
    h = jnp.maximum(h, 0.0)
    h = jnp.maximum(jnp.dot(h, W2, preferred_element_type=jnp.float32) + b2, 0.0)
    ctx = _ln_rows(jnp.dot(h, Wp, preferred_element_type=jnp.float32) + bp)

    zsq = jnp.sum(z * z, axis=1, keepdims=True)
    csq = jnp.sum(cb * cb, axis=1)[None, :]
    d = zsq + csq - 2.0 * jnp.dot(z, cb.T, preferred_element_type=jnp.float32)
    logits = _ln_rows(-jnp.clip(d, 0.0, 10000.0))

    total = logits + CTX * ctx
    idx = jnp.argmax(total, axis=1)
    idx_pure = jnp.argmax(logits, axis=1)

    onehot = (jax.lax.broadcasted_iota(jnp.int32, (z.shape[0], k), 1)
              == idx[:, None]).astype(jnp.float32)
    zq = jnp.dot(onehot, cb, preferred_element_type=jnp.float32)
    loss = (1.0 + COMMIT) * jnp.mean((zq - z) ** 2, axis=1)
    flags = (idx != idx_pure).astype(jnp.float32)
    divpart = jnp.sum(flags.reshape(-1, 128), axis=0)[None, :]
    return zq, loss, idx.astype(jnp.int32), divpart


def _vq_kernel(zf_ref, zs_ref, cbsyn_ref, cbsem_ref,
               syn_W1, syn_b1, syn_g1, syn_be1, syn_W2, syn_b2, syn_Wp, syn_bp,
               sem_W1, sem_b1, sem_g1, sem_be1, sem_W2, sem_b2, sem_Wp, sem_bp,
               zq_syn_ref, zq_sem_ref, loss_syn_ref, loss_sem_ref,
               idx_syn_ref, idx_sem_ref, div_syn_ref, div_sem_ref):
    zq_s, loss_s, idx_s, cnt_s = _branch(
        zf_ref[...], cbsyn_ref[...],
        syn_W1[...], syn_b1[...], syn_g1[...], syn_be1[...],
        syn_W2[...], syn_b2[...], syn_Wp[...], syn_bp[...], N_SYN)
    zq_m, loss_m, idx_m, cnt_m = _branch(
        zs_ref[...], cbsem_ref[...],
        sem_W1[...], sem_b1[...], sem_g1[...], sem_be1[...],
        sem_W2[...], sem_b2[...], sem_Wp[...], sem_bp[...], N_SEM)

    zq_syn_ref[...] = zq_s
    zq_sem_ref[...] = zq_m
    loss_syn_ref[...] = loss_s
    loss_sem_ref[...] = loss_m
    idx_syn_ref[...] = idx_s
    idx_sem_ref[...] = idx_m

    @pl.when(pl.program_id(0) == 0)
    def _():
        div_syn_ref[...] = jnp.zeros_like(div_syn_ref)
        div_sem_ref[...] = jnp.zeros_like(div_sem_ref)

    div_syn_ref[...] += cnt_s * (1.0 / N)
    div_sem_ref[...] += cnt_m * (1.0 / N)


def kernel(z_fast, z_slow, cb_syn, cb_sem,
           syn_W1, syn_b1, syn_g1, syn_be1, syn_W2, syn_b2, syn_Wp, syn_bp,
           sem_W1, sem_b1, sem_g1, sem_be1, sem_W2, sem_b2, sem_Wp, sem_bp,
           update_graph):
    grid = (N // BLK,)
    row_spec = pl.BlockSpec((BLK, DIM), lambda i: (i, 0))
    full = lambda shape: pl.BlockSpec(shape, lambda i: (0,) * len(shape))

    in_specs = [
        row_spec, row_spec,
        full((N_SYN, DIM)), full((N_SEM, DIM)),
        full((DIM, DIM)), full((DIM,)), full((DIM,)), full((DIM,)),
        full((DIM, DIM // 2)), full((DIM // 2,)),
        full((DIM // 2, N_SYN)), full((N_SYN,)),
        full((DIM, DIM)), full((DIM,)), full((DIM,)), full((DIM,)),
        full((DIM, DIM // 2)), full((DIM // 2,)),
        full((DIM // 2, N_SEM)), full((N_SEM,)),
    ]
    out_specs = [
        row_spec, row_spec,
        pl.BlockSpec((BLK,), lambda i: (i,)),
        pl.BlockSpec((BLK,), lambda i: (i,)),
        pl.BlockSpec((BLK,), lambda i: (i,)),
        pl.BlockSpec((BLK,), lambda i: (i,)),
        pl.BlockSpec((1, 128), lambda i: (0, 0)),
        pl.BlockSpec((1, 128), lambda i: (0, 0)),
    ]
    out_shapes = [
        jax.ShapeDtypeStruct((N, DIM), jnp.float32),
        jax.ShapeDtypeStruct((N, DIM), jnp.float32),
        jax.ShapeDtypeStruct((N,), jnp.float32),
        jax.ShapeDtypeStruct((N,), jnp.float32),
        jax.ShapeDtypeStruct((N,), jnp.int32),
        jax.ShapeDtypeStruct((N,), jnp.int32),
        jax.ShapeDtypeStruct((1, 128), jnp.float32),
        jax.ShapeDtypeStruct((1, 128), jnp.float32),
    ]

    (zq_syn, zq_sem, loss_syn, loss_sem, idx_syn, idx_sem,
     div_syn, div_sem) = pl.pallas_call(
        _vq_kernel,
        grid=grid,
        in_specs=in_specs,
        out_specs=out_specs,
        out_shape=out_shapes,
    )(z_fast, z_slow, cb_syn, cb_sem,
      syn_W1, syn_b1, syn_g1, syn_be1, syn_W2, syn_b2, syn_Wp, syn_bp,
      sem_W1, sem_b1, sem_g1, sem_be1, sem_W2, sem_b2, sem_Wp, sem_bp)

    return (zq_syn, zq_sem, loss_syn, loss_sem, idx_syn, idx_sem,
            jnp.sum(div_syn), jnp.sum(div_sem))
